# baseline (speedup 1.0000x reference)
import jax
import jax.numpy as jnp
from jax import lax
from jax.experimental import pallas as pl
from jax.experimental.pallas import tpu as pltpu
from jax.experimental.pallas import tpu_sc as plsc

BATCH = 16384
EMB = 64

_NC = 2
_NS = 16
_NW = _NC * _NS
_BPW = BATCH // _NW


def _sc_gather_body(uid_hbm, iid_hbm, ut_hbm, it_hbm, uf_hbm, vf_hbm,
                    uidx_v, iidx_v, sem):
    wid = lax.axis_index("s") * _NC + lax.axis_index("c")
    base = wid * _BPW
    pltpu.sync_copy(uid_hbm.at[pl.ds(base, _BPW)], uidx_v)
    pltpu.sync_copy(iid_hbm.at[pl.ds(base, _BPW)], iidx_v)

    def body(g, carry):
        off = g * 16
        uvec = uidx_v[pl.ds(off, 16)]
        ivec = iidx_v[pl.ds(off, 16)]
        for k in range(16):
            pltpu.async_copy(ut_hbm.at[pl.ds(uvec[k], 1)],
                             uf_hbm.at[pl.ds(base + off + k, 1)], sem)
            pltpu.async_copy(it_hbm.at[pl.ds(ivec[k], 1)],
                             vf_hbm.at[pl.ds(base + off + k, 1)], sem)
        return carry

    lax.fori_loop(0, _BPW // 16, body, 0)
    pltpu.make_async_copy(
        ut_hbm.at[pl.ds(0, _BPW)], uf_hbm.at[pl.ds(base, _BPW)], sem).wait()
    pltpu.make_async_copy(
        it_hbm.at[pl.ds(0, _BPW)], vf_hbm.at[pl.ds(base, _BPW)], sem).wait()


@jax.jit
def _sc_gather(u_id, i_id, user_table, item_table):
    mesh = plsc.VectorSubcoreMesh(core_axis_name="c", subcore_axis_name="s")
    f = pl.kernel(
        _sc_gather_body,
        out_type=(
            jax.ShapeDtypeStruct((BATCH, EMB), jnp.float32),
            jax.ShapeDtypeStruct((BATCH, EMB), jnp.float32),
        ),
        mesh=mesh,
        scratch_types=[
            pltpu.VMEM((_BPW,), jnp.int32),
            pltpu.VMEM((_BPW,), jnp.int32),
            pltpu.SemaphoreType.DMA,
        ],
    )
    return f(u_id, i_id, user_table, item_table)


def _mlp_body(uf, vf, w0u, w0v, b0, w1, b1, w2, b2, w3, b3, out):
    h = uf[...] @ w0u[...] + vf[...] @ w0v[...] + b0[...]
    h = jnp.maximum(h, 0.0)
    h = jnp.maximum(h @ w1[...] + b1[...], 0.0)
    h = jnp.maximum(h @ w2[...] + b2[...], 0.0)
    out[...] = jnp.sum(h * w3[...], axis=1, keepdims=True) + b3[...]


_BLK = 2048


@jax.jit
def _mlp(uf, vf, w0u, w0v, b0, w1, b1, w2, b2, w3, b3):
    nblk = BATCH // _BLK
    bcast = lambda i: (0, 0)
    return pl.pallas_call(
        _mlp_body,
        grid=(nblk,),
        in_specs=[
            pl.BlockSpec((_BLK, EMB), lambda i: (i, 0)),
            pl.BlockSpec((_BLK, EMB), lambda i: (i, 0)),
            pl.BlockSpec((EMB, 32), bcast),
            pl.BlockSpec((EMB, 32), bcast),
            pl.BlockSpec((1, 32), bcast),
            pl.BlockSpec((32, 16), bcast),
            pl.BlockSpec((1, 16), bcast),
            pl.BlockSpec((16, 8), bcast),
            pl.BlockSpec((1, 8), bcast),
            pl.BlockSpec((1, 8), bcast),
            pl.BlockSpec((1, 1), bcast),
        ],
        out_specs=pl.BlockSpec((_BLK, 1), lambda i: (i, 0)),
        out_shape=jax.ShapeDtypeStruct((BATCH, 1), jnp.float32),
    )(uf, vf, w0u, w0v, b0, w1, b1, w2, b2, w3, b3)


def kernel(u_id, i_id, user_table, item_table, W0, b0, W1, b1, W2, b2, W3, b3):
    uf, vf = _sc_gather(u_id.astype(jnp.int32), i_id.astype(jnp.int32),
                        user_table, item_table)
    out = _mlp(
        uf, vf,
        W0[:EMB], W0[EMB:], b0.reshape(1, -1),
        W1, b1.reshape(1, -1),
        W2, b2.reshape(1, -1),
        W3.reshape(1, -1), b3.reshape(1, 1),
    )
    return out[:, 0]


# speedup vs baseline: 1.6738x; 1.6738x over previous
import jax
import jax.numpy as jnp
from jax import lax
from jax.experimental import pallas as pl
from jax.experimental.pallas import tpu as pltpu
from jax.experimental.pallas import tpu_sc as plsc

BATCH = 16384
EMB = 64

_NC = 2
_NS = 16
_NW = _NC * _NS
_BPW = BATCH // _NW


def _sc_gather_body(uid_hbm, iid_hbm, ut_hbm, it_hbm, uf_hbm, vf_hbm,
                    uidx_v, iidx_v, rows_v, sem):
    wid = lax.axis_index("s") * _NC + lax.axis_index("c")
    base = wid * _BPW
    pltpu.sync_copy(uid_hbm.at[pl.ds(base, _BPW)], uidx_v)
    pltpu.sync_copy(iid_hbm.at[pl.ds(base, _BPW)], iidx_v)

    for tbl_hbm, idx_v, out_hbm in ((ut_hbm, uidx_v, uf_hbm),
                                    (it_hbm, iidx_v, vf_hbm)):
        def body(g, carry, tbl_hbm=tbl_hbm, idx_v=idx_v):
            off = g * 16
            vec = idx_v[pl.ds(off, 16)]
            for k in range(16):
                pltpu.async_copy(tbl_hbm.at[pl.ds(vec[k], 1)],
                                 rows_v.at[pl.ds(off + k, 1)], sem)
            return carry

        lax.fori_loop(0, _BPW // 16, body, 0)
        pltpu.make_async_copy(
            tbl_hbm.at[pl.ds(0, _BPW)], rows_v, sem).wait()
        pltpu.sync_copy(rows_v, out_hbm.at[pl.ds(base, _BPW)])


@jax.jit
def _sc_gather(u_id, i_id, user_table, item_table):
    mesh = plsc.VectorSubcoreMesh(core_axis_name="c", subcore_axis_name="s")
    f = pl.kernel(
        _sc_gather_body,
        out_type=(
            jax.ShapeDtypeStruct((BATCH, EMB), jnp.float32),
            jax.ShapeDtypeStruct((BATCH, EMB), jnp.float32),
        ),
        mesh=mesh,
        scratch_types=[
            pltpu.VMEM((_BPW,), jnp.int32),
            pltpu.VMEM((_BPW,), jnp.int32),
            pltpu.VMEM((_BPW, EMB), jnp.float32),
            pltpu.SemaphoreType.DMA,
        ],
    )
    return f(u_id, i_id, user_table, item_table)


def _mlp_body(uf, vf, w0u, w0v, b0, w1, b1, w2, b2, w3, b3, out):
    h = uf[...] @ w0u[...] + vf[...] @ w0v[...] + b0[...]
    h = jnp.maximum(h, 0.0)
    h = jnp.maximum(h @ w1[...] + b1[...], 0.0)
    h = jnp.maximum(h @ w2[...] + b2[...], 0.0)
    out[...] = jnp.sum(h * w3[...], axis=1, keepdims=True) + b3[...]


_BLK = 2048


@jax.jit
def _mlp(uf, vf, w0u, w0v, b0, w1, b1, w2, b2, w3, b3):
    nblk = BATCH // _BLK
    bcast = lambda i: (0, 0)
    row = lambda i: (i, 0)
    return pl.pallas_call(
        _mlp_body,
        grid=(nblk,),
        in_specs=[
            pl.BlockSpec((_BLK, EMB), row),
            pl.BlockSpec((_BLK, EMB), row),
            pl.BlockSpec((EMB, 32), bcast),
            pl.BlockSpec((EMB, 32), bcast),
            pl.BlockSpec((1, 32), bcast),
            pl.BlockSpec((32, 16), bcast),
            pl.BlockSpec((1, 16), bcast),
            pl.BlockSpec((16, 8), bcast),
            pl.BlockSpec((1, 8), bcast),
            pl.BlockSpec((1, 8), bcast),
            pl.BlockSpec((1, 1), bcast),
        ],
        out_specs=pl.BlockSpec((_BLK, 1), row),
        out_shape=jax.ShapeDtypeStruct((BATCH, 1), jnp.float32),
    )(uf, vf, w0u, w0v, b0, w1, b1, w2, b2, w3, b3)


def kernel(u_id, i_id, user_table, item_table, W0, b0, W1, b1, W2, b2, W3, b3):
    uf, vf = _sc_gather(u_id.astype(jnp.int32), i_id.astype(jnp.int32),
                        user_table, item_table)
    out = _mlp(
        uf, vf,
        W0[:EMB], W0[EMB:], b0.reshape(1, -1),
        W1, b1.reshape(1, -1),
        W2, b2.reshape(1, -1),
        W3.reshape(1, -1), b3.reshape(1, 1),
    )
    return out[:, 0]
